# baseline (speedup 1.0000x reference)
import functools

import jax
import jax.numpy as jnp
from jax import lax
from jax.experimental import pallas as pl
from jax.experimental.pallas import tpu as pltpu
from jax.experimental.pallas import tpu_sc as plsc

B = 1024
NNEG = 256
DIM = 128
LANES = 16
NCHUNK = 128
CPR = NNEG // NCHUNK
NBUF = 4
RPG = NBUF // CPR
NW = 32
BPW = B // NW
KREG = DIM // LANES
TOTAL = BPW * CPR
NGRP = TOTAL // NBUF
TAIL = TOTAL - NGRP * NBUF


def _sc_body(heads_hbm, rels_hbm, negs_hbm, ent_hbm, rel_hbm, out_hbm,
             heads_v, rels_v, negs_v, hrow_v, rrow_v, out_v,
             *bufs_and_sems):
    bufs = bufs_and_sems[:NBUF]
    sem_h, sem_r = bufs_and_sems[NBUF], bufs_and_sems[NBUF + 1]
    sems = bufs_and_sems[NBUF + 2:]

    wid = lax.axis_index("s") * 2 + lax.axis_index("c")
    base = wid * BPW

    pltpu.sync_copy(heads_hbm.at[pl.ds(base, BPW)], heads_v)
    pltpu.sync_copy(rels_hbm.at[pl.ds(base, BPW)], rels_v)
    pltpu.sync_copy(negs_hbm.at[pl.ds(base, BPW)], negs_v)

    cp_h = pltpu.async_copy(ent_hbm.at[heads_v], hrow_v, sem_h)
    cp_r = pltpu.async_copy(rel_hbm.at[rels_v], rrow_v, sem_r)
    for s in range(NBUF):
        pltpu.async_copy(ent_hbm.at[negs_v.at[s // CPR, s % CPR]],
                         bufs[s], sems[s])
    cp_h.wait()
    cp_r.wait()

    def compute_chunk(j, c, buf):
        h = [hrow_v[j, pl.ds(k * LANES, LANES)] *
             rrow_v[j, pl.ds(k * LANES, LANES)] for k in range(KREG)]
        lane = lax.iota(jnp.int32, LANES)

        dnums = lax.GatherDimensionNumbers(
            offset_dims=(), collapsed_slice_dims=(0,), start_index_map=(0,))

        def permute(v, p):
            return lax.gather(v, p[:, None], dnums, slice_sizes=(1,),
                              mode=lax.GatherScatterMode.PROMISE_IN_BOUNDS)

        def rowdot(n):
            acc = buf[n, pl.ds(0, LANES)] * h[0]
            for k in range(1, KREG):
                acc = acc + buf[n, pl.ds(k * LANES, LANES)] * h[k]
            return acc

        low = lane < 8
        lane7 = lane & 7

        def grp_body(g, _):
            r = jnp.zeros((LANES,), jnp.float32)
            for t in range(LANES // 2):
                sa = rowdot(g * LANES + t)
                sb = rowdot(g * LANES + 8 + t)
                sa = sa + permute(sa, lane ^ 8)
                sb = sb + permute(sb, lane ^ 8)
                m = jnp.where(low, sa, sb)
                for step in (4, 2, 1):
                    m = m + permute(m, lane ^ step)
                r = jnp.where(lane7 == t, m, r)
            out_v[j, pl.ds(c * NCHUNK + g * LANES, LANES)] = r
            return 0

        lax.fori_loop(0, NCHUNK // LANES, grp_body, 0)

    def ring_body(q, _):
        for s in range(NBUF):
            j = q * RPG + s // CPR
            c = s % CPR
            pltpu.make_async_copy(ent_hbm.at[negs_v.at[j, c]],
                                  bufs[s], sems[s]).wait()
            compute_chunk(j, c, bufs[s])

            if s < TAIL:
                pltpu.async_copy(ent_hbm.at[negs_v.at[j + RPG, c]],
                                 bufs[s], sems[s])
            else:
                @pl.when(q < NGRP - 1)
                def _():
                    pltpu.async_copy(ent_hbm.at[negs_v.at[j + RPG, c]],
                                     bufs[s], sems[s])
        return 0

    lax.fori_loop(0, NGRP, ring_body, 0)

    for s in range(TAIL):
        j = NGRP * RPG + s // CPR
        c = s % CPR
        pltpu.make_async_copy(ent_hbm.at[negs_v.at[j, c]],
                              bufs[s], sems[s]).wait()
        compute_chunk(j, c, bufs[s])

    pltpu.sync_copy(out_v, out_hbm.at[pl.ds(base, BPW)])


@functools.partial(jax.jit, static_argnames=())
def _launch(heads, rels, negs3, entity_embedding, relation_embedding):
    mesh = plsc.VectorSubcoreMesh(core_axis_name="c", subcore_axis_name="s")
    return pl.kernel(
        _sc_body,
        out_type=jax.ShapeDtypeStruct((B, NNEG), jnp.float32),
        mesh=mesh,
        scratch_types=(
            [
                pltpu.VMEM((BPW,), jnp.int32),
                pltpu.VMEM((BPW,), jnp.int32),
                pltpu.VMEM((BPW, CPR, NCHUNK), jnp.int32),
                pltpu.VMEM((BPW, DIM), jnp.float32),
                pltpu.VMEM((BPW, DIM), jnp.float32),
                pltpu.VMEM((BPW, NNEG), jnp.float32),
            ]
            + [pltpu.VMEM((NCHUNK, DIM), jnp.float32)] * NBUF
            + [pltpu.SemaphoreType.DMA] * (2 + NBUF)
        ),
    )(heads, rels, negs3, entity_embedding, relation_embedding)


def kernel(triples, negs, entity_embedding, relation_embedding):
    heads = triples[:, 0].astype(jnp.int32)
    rels = triples[:, 1].astype(jnp.int32)
    negs3 = negs.astype(jnp.int32).reshape(B, CPR, NCHUNK)
    return _launch(heads, rels, negs3,
                   entity_embedding.astype(jnp.float32),
                   relation_embedding.astype(jnp.float32))


# speedup vs baseline: 1.0544x; 1.0544x over previous
import functools

import jax
import jax.numpy as jnp
from jax import lax
from jax.experimental import pallas as pl
from jax.experimental.pallas import tpu as pltpu
from jax.experimental.pallas import tpu_sc as plsc

B = 1024
NNEG = 256
DIM = 128
LANES = 16
NCHUNK = 128
CPR = NNEG // NCHUNK
NBUF = 4
NW = 32
BPW = B // NW
KREG = DIM // LANES
TOTAL = BPW * CPR


def _sc_body(heads_hbm, rels_hbm, negs_hbm, ent_hbm, rel_hbm, out_hbm,
             heads_v, rels_v, negs_v, hrow_v, rrow_v, out_v, buf_v,
             sem_h, sem_r, sems):
    wid = lax.axis_index("s") * 2 + lax.axis_index("c")
    base = wid * BPW

    pltpu.sync_copy(heads_hbm.at[pl.ds(base, BPW)], heads_v)
    pltpu.sync_copy(rels_hbm.at[pl.ds(base, BPW)], rels_v)
    pltpu.sync_copy(negs_hbm.at[pl.ds(base, BPW)], negs_v)

    cp_h = pltpu.async_copy(ent_hbm.at[heads_v], hrow_v, sem_h)
    cp_r = pltpu.async_copy(rel_hbm.at[rels_v], rrow_v, sem_r)
    for s in range(NBUF):
        pltpu.async_copy(ent_hbm.at[negs_v.at[s // CPR, s % CPR]],
                         buf_v.at[s], sems.at[s])
    cp_h.wait()
    cp_r.wait()

    def compute_chunk(j, c, buf):
        h = [hrow_v[j, pl.ds(k * LANES, LANES)] *
             rrow_v[j, pl.ds(k * LANES, LANES)] for k in range(KREG)]
        lane = lax.iota(jnp.int32, LANES)

        dnums = lax.GatherDimensionNumbers(
            offset_dims=(), collapsed_slice_dims=(0,), start_index_map=(0,))

        def lanesum(v):
            for step in (8, 4, 2, 1):
                perm = lane ^ step
                v = v + lax.gather(v, perm[:, None], dnums, slice_sizes=(1,),
                                   mode=lax.GatherScatterMode.PROMISE_IN_BOUNDS)
            return v

        def grp_body(g, _):
            r = jnp.zeros((LANES,), jnp.float32)
            for i in range(LANES):
                n = g * LANES + i
                acc = buf[n, pl.ds(0, LANES)] * h[0]
                for k in range(1, KREG):
                    acc = acc + buf[n, pl.ds(k * LANES, LANES)] * h[k]
                r = jnp.where(lane == i, lanesum(acc), r)
            out_v[j, pl.ds(c * NCHUNK + g * LANES, LANES)] = r
            return 0

        lax.fori_loop(0, NCHUNK // LANES, grp_body, 0)

    def ring_body(k, _):
        s = k % NBUF
        j = k // CPR
        c = k % CPR
        pltpu.make_async_copy(ent_hbm.at[negs_v.at[j, c]],
                              buf_v.at[s], sems.at[s]).wait()
        compute_chunk(j, c, buf_v.at[s])

        kn = k + NBUF

        @pl.when(kn < TOTAL)
        def _():
            pltpu.async_copy(ent_hbm.at[negs_v.at[kn // CPR, kn % CPR]],
                             buf_v.at[s], sems.at[s])

        return 0

    lax.fori_loop(0, TOTAL, ring_body, 0)

    pltpu.sync_copy(out_v, out_hbm.at[pl.ds(base, BPW)])


@functools.partial(jax.jit, static_argnames=())
def _launch(heads, rels, negs3, entity_embedding, relation_embedding):
    mesh = plsc.VectorSubcoreMesh(core_axis_name="c", subcore_axis_name="s")
    return pl.kernel(
        _sc_body,
        out_type=jax.ShapeDtypeStruct((B, NNEG), jnp.float32),
        mesh=mesh,
        scratch_types=[
            pltpu.VMEM((BPW,), jnp.int32),
            pltpu.VMEM((BPW,), jnp.int32),
            pltpu.VMEM((BPW, CPR, NCHUNK), jnp.int32),
            pltpu.VMEM((BPW, DIM), jnp.float32),
            pltpu.VMEM((BPW, DIM), jnp.float32),
            pltpu.VMEM((BPW, NNEG), jnp.float32),
            pltpu.VMEM((NBUF, NCHUNK, DIM), jnp.float32),
            pltpu.SemaphoreType.DMA,
            pltpu.SemaphoreType.DMA,
            pltpu.SemaphoreType.DMA((NBUF,)),
        ],
    )(heads, rels, negs3, entity_embedding, relation_embedding)


def kernel(triples, negs, entity_embedding, relation_embedding):
    heads = triples[:, 0].astype(jnp.int32)
    rels = triples[:, 1].astype(jnp.int32)
    negs3 = negs.astype(jnp.int32).reshape(B, CPR, NCHUNK)
    return _launch(heads, rels, negs3,
                   entity_embedding.astype(jnp.float32),
                   relation_embedding.astype(jnp.float32))
